# baseline (speedup 1.0000x reference)
import functools

import jax
import jax.numpy as jnp
from jax import lax
from jax.experimental import pallas as pl
from jax.experimental.pallas import tpu as pltpu
from jax.experimental.pallas import tpu_sc as plsc

B, C, H, W = 2, 96, 384, 384
WP = 392
HP = 400
MP = HP * WP
MOUT = H * WP
MBLK = 6272
NBLK = MOUT // MBLK
HW = H * W
NMAPS = B * C


def _conv_tap_body(w_ref, xlo_ref, xhi_ref, out_ref, *, cout):
    xcat = jnp.concatenate([xlo_ref[0], xhi_ref[0]], axis=1)
    acc = jnp.zeros((cout, MBLK), jnp.float32)
    for t in range(9):
        s = (t // 3) * WP + (t % 3)
        acc = acc + lax.dot_general(
            w_ref[t], xcat[:, s:s + MBLK],
            (((1,), (0,)), ((), ())), preferred_element_type=jnp.float32)
    out_ref[0] = acc


def _conv1(xp_bf16, w1):
    return pl.pallas_call(
        functools.partial(_conv_tap_body, cout=2 * C),
        grid=(B, NBLK),
        in_specs=[
            pl.BlockSpec((9, 2 * C, C), lambda b, i: (0, 0, 0)),
            pl.BlockSpec((1, C, MBLK), lambda b, i: (b, 0, i)),
            pl.BlockSpec((1, C, MBLK), lambda b, i: (b, 0, i + 1)),
        ],
        out_specs=pl.BlockSpec((1, 2 * C, MBLK), lambda b, i: (b, 0, i)),
        out_shape=jax.ShapeDtypeStruct((B, 2 * C, MOUT), jnp.float32),
    )(w1, xp_bf16, xp_bf16)


def _conv2_body(w_ref, b_ref, xlo_ref, xhi_ref, out_ref):
    xcat = jnp.concatenate(
        [xlo_ref[0].astype(jnp.bfloat16), xhi_ref[0].astype(jnp.bfloat16)],
        axis=1)
    acc = jnp.zeros((C, MBLK), jnp.float32)
    for t in range(9):
        s = (t // 3) * WP + (t % 3)
        acc = acc + lax.dot_general(
            w_ref[t], xcat[:, s:s + MBLK],
            (((1,), (0,)), ((), ())), preferred_element_type=jnp.float32)
    acc = acc + b_ref[...]
    for r in range(16):
        out_ref[0, :, r * W:(r + 1) * W] = acc[:, r * WP:r * WP + W]


def _conv2(mp_f32, w2, b2):
    return pl.pallas_call(
        _conv2_body,
        grid=(B, NBLK),
        in_specs=[
            pl.BlockSpec((9, C, C), lambda b, i: (0, 0, 0)),
            pl.BlockSpec((C, 1), lambda b, i: (0, 0)),
            pl.BlockSpec((1, C, MBLK), lambda b, i: (b, 0, i)),
            pl.BlockSpec((1, C, MBLK), lambda b, i: (b, 0, i + 1)),
        ],
        out_specs=pl.BlockSpec((1, C, 16 * W), lambda b, i: (b, 0, i)),
        out_shape=jax.ShapeDtypeStruct((B, C, HW), jnp.float32),
    )(w2, b2, mp_f32, mp_f32)


_MAPS_PER_TILE = NMAPS // 32
_RB = 8
_KB = H // _RB
_SPAN = 2 * _RB * WP
_OUTB = _RB * WP


def _sc_body(offs_hbm, xmap_hbm, out_hbm, map_buf, span_buf, out_buf, zbuf,
             si, so):
    cid = lax.axis_index("c")
    sid = lax.axis_index("s")
    wid = sid * 2 + cid
    lane = lax.iota(jnp.int32, 16)
    lane2 = lane * 2
    lane_f = lane.astype(jnp.float32)

    def _zero(i, _):
        out_buf[pl.ds(i * 16, 16)] = jnp.zeros((16,), jnp.float32)
        return 0
    lax.fori_loop(0, 2 * _OUTB // 16, _zero, 0)

    def _zero2(i, _):
        zbuf[pl.ds(i * 16, 16)] = jnp.zeros((16,), jnp.float32)
        return 0
    lax.fori_loop(0, _OUTB // 16, _zero2, 0)

    def map_body(mi, _):
        g = wid * _MAPS_PER_TILE + mi
        b = g // C
        c = g - b * C
        ch_base = b * (2 * C) + 2 * c
        obase = g * MP
        pltpu.sync_copy(xmap_hbm.at[pl.ds(g * (HW // 2), HW // 2)], map_buf)
        pltpu.sync_copy(zbuf.at[pl.ds(0, WP)],
                        out_hbm.at[pl.ds(obase, WP)])
        pltpu.sync_copy(zbuf.at[pl.ds(0, _OUTB)],
                        out_hbm.at[pl.ds(obase + (H + 1) * WP, _OUTB)])

        def _in_copy(k):
            half = jnp.where(k >= _KB // 2, 1, 0)
            ch = ch_base + half
            kk = k - half * (_KB // 2)
            pb = k & 1
            return pltpu.make_async_copy(
                offs_hbm.at[pl.ds(ch * MOUT + kk * _SPAN, _SPAN)],
                span_buf.at[pl.ds(pb * _SPAN, _SPAN)], si.at[pb])

        def _out_copy(k):
            pb = k & 1
            return pltpu.make_async_copy(
                out_buf.at[pl.ds(pb * _OUTB, _OUTB)],
                out_hbm.at[pl.ds(obase + (k * _RB + 1) * WP, _OUTB)],
                so.at[pb])

        _in_copy(0).start()

        def blk_body(k, _):
            pb = k & 1

            @pl.when(k < _KB - 1)
            def _():
                _in_copy(k + 1).start()

            _in_copy(k).wait()

            @pl.when(k >= 2)
            def _():
                _out_copy(k - 2).wait()

            sbase = pb * _SPAN
            ob = pb * _OUTB

            def row_body(r, _):
                h_f = (k * _RB + r).astype(jnp.float32)
                rbase = sbase + r * (2 * WP)
                for j in range(24):
                    iy = lane2 + (rbase + 32 * j + (8 if j >= 12 else 0))
                    y = plsc.load_gather(span_buf, [iy])
                    x = plsc.load_gather(span_buf, [iy + 1])
                    cy = jnp.minimum(jnp.maximum(y + h_f, 0.0),
                                     jnp.float32(H - 1))
                    cx = jnp.minimum(jnp.maximum(x + (lane_f + (16.0 * j)),
                                                 0.0), jnp.float32(W - 1))
                    y0 = cy.astype(jnp.int32)
                    x0 = cx.astype(jnp.int32)
                    dy = cy - y0.astype(jnp.float32)
                    dx = cx - x0.astype(jnp.float32)
                    base = y0 * W + x0
                    lim = jnp.int32(HW - 1)

                    def corner(ci):
                        wd = plsc.load_gather(
                            map_buf, [lax.shift_right_logical(ci, 1)])
                        odd = (ci & 1) == 1
                        bits = jnp.where(odd, wd & jnp.int32(-65536),
                                         lax.shift_left(wd, 16))
                        return plsc.bitcast(bits, jnp.float32)

                    v00 = corner(base)
                    v01 = corner(jnp.minimum(base + 1, lim))
                    v10 = corner(jnp.minimum(base + W, lim))
                    v11 = corner(jnp.minimum(base + (W + 1), lim))
                    vt = dy * (v10 - v00) + v00
                    vb = dy * (v11 - v01) + v01
                    mp = dx * (vb - vt) + vt
                    sidx = lane + (ob + r * WP + (1 + 16 * j))
                    plsc.store_scatter(out_buf, [sidx], mp)
                return 0

            lax.fori_loop(0, _RB, row_body, 0)
            _out_copy(k).start()
            return 0

        lax.fori_loop(0, _KB, blk_body, 0)
        _out_copy(_KB - 2).wait()
        _out_copy(_KB - 1).wait()
        return 0

    lax.fori_loop(0, _MAPS_PER_TILE, map_body, 0)


def _sc_bilinear(offs_flat, xmap):
    mesh = plsc.VectorSubcoreMesh(core_axis_name="c", subcore_axis_name="s")
    f = pl.kernel(
        _sc_body,
        out_type=jax.ShapeDtypeStruct((NMAPS * MP,), jnp.float32),
        mesh=mesh,
        compiler_params=pltpu.CompilerParams(needs_layout_passes=False),
        scratch_types=[
            pltpu.VMEM((HW // 2,), jnp.int32),
            pltpu.VMEM((2 * _SPAN,), jnp.float32),
            pltpu.VMEM((2 * _OUTB,), jnp.float32),
            pltpu.VMEM((_OUTB,), jnp.float32),
            pltpu.SemaphoreType.DMA((2,)),
            pltpu.SemaphoreType.DMA((2,)),
        ],
    )
    return f(offs_flat, xmap)


def kernel(x, W_off, W_conv, b_conv):
    xp = jnp.pad(x, ((0, 0), (0, 0), (1, 15), (1, 7)))
    xp = xp.astype(jnp.bfloat16).reshape(B, C, MP)
    w1 = W_off.transpose(2, 3, 0, 1).reshape(9, 2 * C, C).astype(jnp.bfloat16)
    w2 = W_conv.transpose(2, 3, 0, 1).reshape(9, C, C).astype(jnp.bfloat16)
    b2 = b_conv.reshape(C, 1)
    xmap = lax.bitcast_convert_type(
        x.astype(jnp.bfloat16).reshape(B, C * (HW // 2), 2), jnp.int32)

    offs = _conv1(xp, w1)
    mapped = _sc_bilinear(offs.reshape(B * 2 * C * MOUT),
                          xmap.reshape(B * C * (HW // 2)))
    out = _conv2(mapped.reshape(B, C, MP), w2, b2)
    return out.reshape(B, C, H, W)


# speedup vs baseline: 3.3811x; 3.3811x over previous
import functools

import jax
import jax.numpy as jnp
from jax import lax
from jax.experimental import pallas as pl
from jax.experimental.pallas import tpu as pltpu
from jax.experimental.pallas import tpu_sc as plsc

B, C, H, W = 2, 96, 384, 384
WP = 392
HP = 400
MP = HP * WP
MOUT = H * WP
MBLK = 6272
NBLK = MOUT // MBLK
HW = H * W
NMAPS = B * C


def _conv_tap_body(w_ref, xlo_ref, xhi_ref, out_ref, *, cout):
    xcat = jnp.concatenate([xlo_ref[0], xhi_ref[0]], axis=1)
    acc = jnp.zeros((cout, MBLK), jnp.float32)
    for t in range(9):
        s = (t // 3) * WP + (t % 3)
        acc = acc + lax.dot_general(
            w_ref[t], xcat[:, s:s + MBLK],
            (((1,), (0,)), ((), ())), preferred_element_type=jnp.float32)
    out_ref[0] = acc


def _conv1(xp_bf16, w1):
    return pl.pallas_call(
        functools.partial(_conv_tap_body, cout=2 * C),
        grid=(B, NBLK),
        in_specs=[
            pl.BlockSpec((9, 2 * C, C), lambda b, i: (0, 0, 0)),
            pl.BlockSpec((1, C, MBLK), lambda b, i: (b, 0, i)),
            pl.BlockSpec((1, C, MBLK), lambda b, i: (b, 0, i + 1)),
        ],
        out_specs=pl.BlockSpec((1, 2 * C, MBLK), lambda b, i: (b, 0, i)),
        out_shape=jax.ShapeDtypeStruct((B, 2 * C, MOUT), jnp.float32),
    )(w1, xp_bf16, xp_bf16)


def _conv2_body(w_ref, b_ref, xlo_ref, xhi_ref, out_ref):
    xcat = jnp.concatenate(
        [xlo_ref[0].astype(jnp.bfloat16), xhi_ref[0].astype(jnp.bfloat16)],
        axis=1)
    acc = jnp.zeros((C, MBLK), jnp.float32)
    for t in range(9):
        s = (t // 3) * WP + (t % 3)
        acc = acc + lax.dot_general(
            w_ref[t], xcat[:, s:s + MBLK],
            (((1,), (0,)), ((), ())), preferred_element_type=jnp.float32)
    acc = acc + b_ref[...]
    for r in range(16):
        out_ref[0, :, r * W:(r + 1) * W] = acc[:, r * WP:r * WP + W]


def _conv2(mp_f32, w2, b2):
    return pl.pallas_call(
        _conv2_body,
        grid=(B, NBLK),
        in_specs=[
            pl.BlockSpec((9, C, C), lambda b, i: (0, 0, 0)),
            pl.BlockSpec((C, 1), lambda b, i: (0, 0)),
            pl.BlockSpec((1, C, MBLK), lambda b, i: (b, 0, i)),
            pl.BlockSpec((1, C, MBLK), lambda b, i: (b, 0, i + 1)),
        ],
        out_specs=pl.BlockSpec((1, C, 16 * W), lambda b, i: (b, 0, i)),
        out_shape=jax.ShapeDtypeStruct((B, C, HW), jnp.float32),
    )(w2, b2, mp_f32, mp_f32)


_MAPS_PER_TILE = NMAPS // 32
_RB = 8
_KB = H // _RB
_SPAN = 2 * _RB * WP
_OUTB = _RB * WP


def _sc_body(offs_hbm, xmap_hbm, out_hbm, map_buf, span_buf, out_buf, zbuf,
             si, so):
    cid = lax.axis_index("c")
    sid = lax.axis_index("s")
    wid = sid * 2 + cid
    lane = lax.iota(jnp.int32, 16)
    lane2 = lane * 2
    lane_f = lane.astype(jnp.float32)

    def _zero(i, _):
        out_buf[pl.ds(i * 16, 16)] = jnp.zeros((16,), jnp.float32)
        return 0
    lax.fori_loop(0, 2 * _OUTB // 16, _zero, 0)

    def _zero2(i, _):
        zbuf[pl.ds(i * 16, 16)] = jnp.zeros((16,), jnp.float32)
        return 0
    lax.fori_loop(0, _OUTB // 16, _zero2, 0)

    def map_body(mi, _):
        g = wid * _MAPS_PER_TILE + mi
        b = g // C
        c = g - b * C
        ch_base = b * (2 * C) + 2 * c
        obase = g * MP
        pltpu.sync_copy(xmap_hbm.at[pl.ds(g * (HW // 2), HW // 2)], map_buf)
        pltpu.sync_copy(zbuf.at[pl.ds(0, WP)],
                        out_hbm.at[pl.ds(obase, WP)])
        pltpu.sync_copy(zbuf.at[pl.ds(0, _OUTB)],
                        out_hbm.at[pl.ds(obase + (H + 1) * WP, _OUTB)])

        def _in_copy(k):
            half = jnp.where(k >= _KB // 2, 1, 0)
            ch = ch_base + half
            kk = k - half * (_KB // 2)
            pb = k & 1
            return pltpu.make_async_copy(
                offs_hbm.at[pl.ds(ch * MOUT + kk * _SPAN, _SPAN)],
                span_buf.at[pl.ds(pb * _SPAN, _SPAN)], si.at[pb])

        def _out_copy(k):
            pb = k & 1
            return pltpu.make_async_copy(
                out_buf.at[pl.ds(pb * _OUTB, _OUTB)],
                out_hbm.at[pl.ds(obase + (k * _RB + 1) * WP, _OUTB)],
                so.at[pb])

        _in_copy(0).start()

        def blk_body(k, _):
            pb = k & 1

            @pl.when(k < _KB - 1)
            def _():
                _in_copy(k + 1).start()

            _in_copy(k).wait()

            @pl.when(k >= 2)
            def _():
                _out_copy(k - 2).wait()

            sbase = pb * _SPAN
            ob = pb * _OUTB

            def row_body(r, _):
                h_f = (k * _RB + r).astype(jnp.float32)
                rbase = sbase + r * (2 * WP)
                for j in range(24):
                    iy = lane2 + (rbase + 32 * j + (8 if j >= 12 else 0))
                    y = plsc.load_gather(span_buf, [iy])
                    x = plsc.load_gather(span_buf, [iy + 1])
                    cy = jnp.minimum(jnp.maximum(y + h_f, 0.0),
                                     jnp.float32(H - 1))
                    cx = jnp.minimum(jnp.maximum(x + (lane_f + (16.0 * j)),
                                                 0.0), jnp.float32(W - 1))
                    y0 = cy.astype(jnp.int32)
                    x0 = cx.astype(jnp.int32)
                    dy = cy - y0.astype(jnp.float32)
                    dx = cx - x0.astype(jnp.float32)
                    base = y0 * W + x0
                    lim = jnp.int32(HW - 1)

                    def corner(ci):
                        wd = plsc.load_gather(
                            map_buf, [lax.shift_right_logical(ci, 1)])
                        odd = (ci & 1) == 1
                        bits = jnp.where(odd, wd & jnp.int32(-65536),
                                         lax.shift_left(wd, 16))
                        return plsc.bitcast(bits, jnp.float32)

                    v00 = corner(base)
                    v01 = corner(jnp.minimum(base + 1, lim))
                    v10 = corner(jnp.minimum(base + W, lim))
                    v11 = corner(jnp.minimum(base + (W + 1), lim))
                    vt = dy * (v10 - v00) + v00
                    vb = dy * (v11 - v01) + v01
                    mp = dx * (vb - vt) + vt
                    sidx = lane + (ob + r * WP + (1 + 16 * j))
                    plsc.store_scatter(out_buf, [sidx], mp)
                return 0

            lax.fori_loop(0, _RB, row_body, 0)
            _out_copy(k).start()
            return 0

        lax.fori_loop(0, _KB, blk_body, 0)
        _out_copy(_KB - 2).wait()
        _out_copy(_KB - 1).wait()
        return 0

    lax.fori_loop(0, _MAPS_PER_TILE, map_body, 0)


def _sc_bilinear(offs_flat, xmap):
    mesh = plsc.VectorSubcoreMesh(core_axis_name="c", subcore_axis_name="s")
    f = pl.kernel(
        _sc_body,
        out_type=jax.ShapeDtypeStruct((NMAPS * MP,), jnp.float32),
        mesh=mesh,
        compiler_params=pltpu.CompilerParams(needs_layout_passes=False),
        scratch_types=[
            pltpu.VMEM((HW // 2,), jnp.int32),
            pltpu.VMEM((2 * _SPAN,), jnp.float32),
            pltpu.VMEM((2 * _OUTB,), jnp.float32),
            pltpu.VMEM((_OUTB,), jnp.float32),
            pltpu.SemaphoreType.DMA((2,)),
            pltpu.SemaphoreType.DMA((2,)),
        ],
    )
    return f(offs_flat, xmap)


def kernel(x, W_off, W_conv, b_conv):
    xp = jnp.pad(x, ((0, 0), (0, 0), (1, 15), (1, 7)))
    xp = xp.astype(jnp.bfloat16).reshape(B, C, MP)
    w1 = W_off.transpose(2, 3, 0, 1).reshape(9, 2 * C, C).astype(jnp.bfloat16)
    w2 = W_conv.transpose(2, 3, 0, 1).reshape(9, C, C).astype(jnp.bfloat16)
    b2 = b_conv.reshape(C, 1)
    xmap = lax.bitcast_convert_type(
        x.astype(jnp.bfloat16).reshape(NMAPS, HW // 2, 2),
        jnp.int32).reshape(NMAPS * (HW // 2))

    offs = _conv1(xp, w1)
    mapped = _sc_bilinear(offs.reshape(B * 2 * C * MOUT), xmap)
    out = _conv2(mapped.reshape(B, C, MP), w2, b2)
    return out.reshape(B, C, H, W)


# speedup vs baseline: 3.5036x; 1.0362x over previous
import functools

import jax
import jax.numpy as jnp
from jax import lax
from jax.experimental import pallas as pl
from jax.experimental.pallas import tpu as pltpu
from jax.experimental.pallas import tpu_sc as plsc

B, C, H, W = 2, 96, 384, 384
WP = 392
HP = 400
MP = HP * WP
MOUT = H * WP
MBLK = 6272
NBLK = MOUT // MBLK
HW = H * W
NMAPS = B * C


def _conv_tap_body(w_ref, xlo_ref, xhi_ref, out_ref, *, cout):
    xcat = jnp.concatenate([xlo_ref[0], xhi_ref[0]], axis=1)
    acc = jnp.zeros((cout, MBLK), jnp.float32)
    for t in range(9):
        s = (t // 3) * WP + (t % 3)
        acc = acc + lax.dot_general(
            w_ref[t], xcat[:, s:s + MBLK],
            (((1,), (0,)), ((), ())), preferred_element_type=jnp.float32)
    out_ref[0] = acc


def _conv1(xp_bf16, w1, b):
    return pl.pallas_call(
        functools.partial(_conv_tap_body, cout=2 * C),
        grid=(NBLK,),
        in_specs=[
            pl.BlockSpec((9, 2 * C, C), lambda i: (0, 0, 0)),
            pl.BlockSpec((1, C, MBLK), lambda i, b=b: (b, 0, i)),
            pl.BlockSpec((1, C, MBLK), lambda i, b=b: (b, 0, i + 1)),
        ],
        out_specs=pl.BlockSpec((1, 2 * C, MBLK), lambda i: (0, 0, i)),
        out_shape=jax.ShapeDtypeStruct((1, 2 * C, MOUT), jnp.float32),
    )(w1, xp_bf16, xp_bf16)


def _conv2_body(w_ref, b_ref, xlo_ref, xhi_ref, out_ref):
    xcat = jnp.concatenate(
        [xlo_ref[0].astype(jnp.bfloat16), xhi_ref[0].astype(jnp.bfloat16)],
        axis=1)
    acc = jnp.zeros((C, MBLK), jnp.float32)
    for t in range(9):
        s = (t // 3) * WP + (t % 3)
        acc = acc + lax.dot_general(
            w_ref[t], xcat[:, s:s + MBLK],
            (((1,), (0,)), ((), ())), preferred_element_type=jnp.float32)
    acc = acc + b_ref[...]
    for r in range(16):
        out_ref[0, :, r, :] = acc[:, r * WP:r * WP + W]


def _conv2(mp_f32, w2, b2):
    return pl.pallas_call(
        _conv2_body,
        grid=(NBLK,),
        in_specs=[
            pl.BlockSpec((9, C, C), lambda i: (0, 0, 0)),
            pl.BlockSpec((C, 1), lambda i: (0, 0)),
            pl.BlockSpec((1, C, MBLK), lambda i: (0, 0, i)),
            pl.BlockSpec((1, C, MBLK), lambda i: (0, 0, i + 1)),
        ],
        out_specs=pl.BlockSpec((1, C, 16, W), lambda i: (0, 0, i, 0)),
        out_shape=jax.ShapeDtypeStruct((1, C, H, W), jnp.float32),
    )(w2, b2, mp_f32, mp_f32)


_MAPS_PER_TILE = C // 32
_RB = 8
_KB = H // _RB
_SPAN = 2 * _RB * WP
_OUTB = _RB * WP


def _sc_body(offs_hbm, xmap_hbm, out_hbm, map_buf, span_buf, out_buf, zbuf,
             si, so):
    cid = lax.axis_index("c")
    sid = lax.axis_index("s")
    wid = sid * 2 + cid
    lane = lax.iota(jnp.int32, 16)
    lane2 = lane * 2
    lane_f = lane.astype(jnp.float32)

    def _zero(i, _):
        out_buf[pl.ds(i * 16, 16)] = jnp.zeros((16,), jnp.float32)
        return 0
    lax.fori_loop(0, 2 * _OUTB // 16, _zero, 0)

    def _zero2(i, _):
        zbuf[pl.ds(i * 16, 16)] = jnp.zeros((16,), jnp.float32)
        return 0
    lax.fori_loop(0, _OUTB // 16, _zero2, 0)

    def map_body(mi, _):
        g = wid * _MAPS_PER_TILE + mi
        ch_base = 2 * g
        obase = g * MP
        pltpu.sync_copy(xmap_hbm.at[pl.ds(g * (HW // 2), HW // 2)], map_buf)
        pltpu.sync_copy(zbuf.at[pl.ds(0, WP)],
                        out_hbm.at[pl.ds(obase, WP)])
        pltpu.sync_copy(zbuf.at[pl.ds(0, _OUTB)],
                        out_hbm.at[pl.ds(obase + (H + 1) * WP, _OUTB)])

        def _in_copy(k):
            half = jnp.where(k >= _KB // 2, 1, 0)
            ch = ch_base + half
            kk = k - half * (_KB // 2)
            pb = k & 1
            return pltpu.make_async_copy(
                offs_hbm.at[pl.ds(ch * MOUT + kk * _SPAN, _SPAN)],
                span_buf.at[pl.ds(pb * _SPAN, _SPAN)], si.at[pb])

        def _out_copy(k):
            pb = k & 1
            return pltpu.make_async_copy(
                out_buf.at[pl.ds(pb * _OUTB, _OUTB)],
                out_hbm.at[pl.ds(obase + (k * _RB + 1) * WP, _OUTB)],
                so.at[pb])

        _in_copy(0).start()

        def blk_body(k, _):
            pb = k & 1

            @pl.when(k < _KB - 1)
            def _():
                _in_copy(k + 1).start()

            _in_copy(k).wait()

            @pl.when(k >= 2)
            def _():
                _out_copy(k - 2).wait()

            sbase = pb * _SPAN
            ob = pb * _OUTB

            def row_body(r, _):
                h_f = (k * _RB + r).astype(jnp.float32)
                rbase = sbase + r * (2 * WP)
                for j in range(24):
                    iy = lane2 + (rbase + 32 * j + (8 if j >= 12 else 0))
                    y = plsc.load_gather(span_buf, [iy])
                    x = plsc.load_gather(span_buf, [iy + 1])
                    cy = jnp.minimum(jnp.maximum(y + h_f, 0.0),
                                     jnp.float32(H - 1))
                    cx = jnp.minimum(jnp.maximum(x + (lane_f + (16.0 * j)),
                                                 0.0), jnp.float32(W - 1))
                    y0 = cy.astype(jnp.int32)
                    x0 = cx.astype(jnp.int32)
                    dy = cy - y0.astype(jnp.float32)
                    dx = cx - x0.astype(jnp.float32)
                    base = y0 * W + x0
                    lim = jnp.int32(HW - 1)

                    def corner(ci):
                        wd = plsc.load_gather(
                            map_buf, [lax.shift_right_logical(ci, 1)])
                        odd = (ci & 1) == 1
                        bits = jnp.where(odd, wd & jnp.int32(-65536),
                                         lax.shift_left(wd, 16))
                        return plsc.bitcast(bits, jnp.float32)

                    v00 = corner(base)
                    v01 = corner(jnp.minimum(base + 1, lim))
                    v10 = corner(jnp.minimum(base + W, lim))
                    v11 = corner(jnp.minimum(base + (W + 1), lim))
                    vt = dy * (v10 - v00) + v00
                    vb = dy * (v11 - v01) + v01
                    mp = dx * (vb - vt) + vt
                    sidx = lane + (ob + r * WP + (1 + 16 * j))
                    plsc.store_scatter(out_buf, [sidx], mp)
                return 0

            lax.fori_loop(0, _RB, row_body, 0)
            _out_copy(k).start()
            return 0

        lax.fori_loop(0, _KB, blk_body, 0)
        _out_copy(_KB - 2).wait()
        _out_copy(_KB - 1).wait()
        return 0

    lax.fori_loop(0, _MAPS_PER_TILE, map_body, 0)


def _sc_bilinear(offs_flat, xmap):
    mesh = plsc.VectorSubcoreMesh(core_axis_name="c", subcore_axis_name="s")
    f = pl.kernel(
        _sc_body,
        out_type=jax.ShapeDtypeStruct((C * MP,), jnp.float32),
        mesh=mesh,
        compiler_params=pltpu.CompilerParams(needs_layout_passes=False),
        scratch_types=[
            pltpu.VMEM((HW // 2,), jnp.int32),
            pltpu.VMEM((2 * _SPAN,), jnp.float32),
            pltpu.VMEM((2 * _OUTB,), jnp.float32),
            pltpu.VMEM((_OUTB,), jnp.float32),
            pltpu.SemaphoreType.DMA((2,)),
            pltpu.SemaphoreType.DMA((2,)),
        ],
    )
    return f(offs_flat, xmap)


def kernel(x, W_off, W_conv, b_conv):
    xp = jnp.pad(x, ((0, 0), (0, 0), (1, 15), (1, 7)))
    xp = xp.astype(jnp.bfloat16).reshape(B, C, MP)
    w1 = W_off.transpose(2, 3, 0, 1).reshape(9, 2 * C, C).astype(jnp.bfloat16)
    w2 = W_conv.transpose(2, 3, 0, 1).reshape(9, C, C).astype(jnp.bfloat16)
    b2 = b_conv.reshape(C, 1)
    xmap = lax.bitcast_convert_type(
        x.astype(jnp.bfloat16).reshape(NMAPS, HW // 2, 2),
        jnp.int32).reshape(NMAPS * (HW // 2))

    outs = []
    for b in range(B):
        offs = _conv1(xp, w1, b)
        xmap_b = lax.slice(xmap, (b * C * (HW // 2),),
                           ((b + 1) * C * (HW // 2),))
        mapped = _sc_bilinear(offs.reshape(2 * C * MOUT), xmap_b)
        outs.append(_conv2(mapped.reshape(1, C, MP), w2, b2))
    return jnp.concatenate(outs, axis=0)


# speedup vs baseline: 3.5048x; 1.0003x over previous
import functools

import jax
import jax.numpy as jnp
from jax import lax
from jax.experimental import pallas as pl
from jax.experimental.pallas import tpu as pltpu
from jax.experimental.pallas import tpu_sc as plsc

B, C, H, W = 2, 96, 384, 384
WP = 392
HP = 400
MP = HP * WP
MOUT = H * WP
MBLK = 6272
NBLK = MOUT // MBLK
HW = H * W
NMAPS = B * C


def _conv_tap_body(w_ref, xlo_ref, xhi_ref, out_ref, *, cout):
    xcat = jnp.concatenate([xlo_ref[0], xhi_ref[0]], axis=1)
    acc = jnp.zeros((cout, MBLK), jnp.float32)
    for t in range(9):
        s = (t // 3) * WP + (t % 3)
        acc = acc + lax.dot_general(
            w_ref[t], xcat[:, s:s + MBLK],
            (((1,), (0,)), ((), ())), preferred_element_type=jnp.float32)
    out_ref[0] = acc


def _conv1(xp_bf16, w1, b):
    return pl.pallas_call(
        functools.partial(_conv_tap_body, cout=2 * C),
        grid=(NBLK,),
        in_specs=[
            pl.BlockSpec((9, 2 * C, C), lambda i: (0, 0, 0)),
            pl.BlockSpec((1, C, MBLK), lambda i, b=b: (b, 0, i)),
            pl.BlockSpec((1, C, MBLK), lambda i, b=b: (b, 0, i + 1)),
        ],
        out_specs=pl.BlockSpec((1, 2 * C, MBLK), lambda i: (0, 0, i)),
        out_shape=jax.ShapeDtypeStruct((1, 2 * C, MOUT), jnp.float32),
    )(w1, xp_bf16, xp_bf16)


def _conv2_body(w_ref, b_ref, xlo_ref, xhi_ref, out_ref):
    xcat = jnp.concatenate(
        [xlo_ref[0].astype(jnp.bfloat16), xhi_ref[0].astype(jnp.bfloat16)],
        axis=1)
    acc = jnp.zeros((C, MBLK), jnp.float32)
    for t in range(9):
        s = (t // 3) * WP + (t % 3)
        acc = acc + lax.dot_general(
            w_ref[t], xcat[:, s:s + MBLK],
            (((1,), (0,)), ((), ())), preferred_element_type=jnp.float32)
    acc = acc + b_ref[...]
    for r in range(16):
        out_ref[0, :, r, :] = acc[:, r * WP:r * WP + W]


def _conv2(mp_f32, w2, b2):
    return pl.pallas_call(
        _conv2_body,
        grid=(NBLK,),
        in_specs=[
            pl.BlockSpec((9, C, C), lambda i: (0, 0, 0)),
            pl.BlockSpec((C, 1), lambda i: (0, 0)),
            pl.BlockSpec((1, C, MBLK), lambda i: (0, 0, i)),
            pl.BlockSpec((1, C, MBLK), lambda i: (0, 0, i + 1)),
        ],
        out_specs=pl.BlockSpec((1, C, 16, W), lambda i: (0, 0, i, 0)),
        out_shape=jax.ShapeDtypeStruct((1, C, H, W), jnp.float32),
    )(w2, b2, mp_f32, mp_f32)


_MAPS_PER_TILE = C // 32
_RB = 8
_KB = H // _RB
_SPAN = 2 * _RB * WP
_OUTB = _RB * WP


def _sc_body(offs_hbm, xmap_hbm, out_hbm, map_buf, span_buf, out_buf, zbuf,
             si, so):
    cid = lax.axis_index("c")
    sid = lax.axis_index("s")
    wid = sid * 2 + cid
    lane = lax.iota(jnp.int32, 16)
    lane2 = lane * 2
    lane_f = lane.astype(jnp.float32)

    def _zero(i, _):
        out_buf[pl.ds(i * 16, 16)] = jnp.zeros((16,), jnp.float32)
        return 0
    lax.fori_loop(0, 2 * _OUTB // 16, _zero, 0)

    def _zero2(i, _):
        zbuf[pl.ds(i * 16, 16)] = jnp.zeros((16,), jnp.float32)
        return 0
    lax.fori_loop(0, _OUTB // 16, _zero2, 0)

    def map_body(mi, _):
        g = wid * _MAPS_PER_TILE + mi
        ch_base = 2 * g
        obase = g * MP
        pltpu.sync_copy(xmap_hbm.at[pl.ds(g * (HW // 2), HW // 2)], map_buf)
        pltpu.sync_copy(zbuf.at[pl.ds(0, WP)],
                        out_hbm.at[pl.ds(obase, WP)])
        pltpu.sync_copy(zbuf.at[pl.ds(0, _OUTB)],
                        out_hbm.at[pl.ds(obase + (H + 1) * WP, _OUTB)])

        def _in_copy(k):
            half = jnp.where(k >= _KB // 2, 1, 0)
            ch = ch_base + half
            kk = k - half * (_KB // 2)
            pb = k & 1
            return pltpu.make_async_copy(
                offs_hbm.at[pl.ds(ch * MOUT + kk * _SPAN, _SPAN)],
                span_buf.at[pl.ds(pb * _SPAN, _SPAN)], si.at[pb])

        def _out_copy(k):
            pb = k & 1
            return pltpu.make_async_copy(
                out_buf.at[pl.ds(pb * _OUTB, _OUTB)],
                out_hbm.at[pl.ds(obase + (k * _RB + 1) * WP, _OUTB)],
                so.at[pb])

        _in_copy(0).start()

        def blk_body(k, _):
            pb = k & 1

            @pl.when(k < _KB - 1)
            def _():
                _in_copy(k + 1).start()

            _in_copy(k).wait()

            @pl.when(k >= 2)
            def _():
                _out_copy(k - 2).wait()

            sbase = pb * _SPAN
            ob = pb * _OUTB

            def row_body(r, _):
                h_f = (k * _RB + r).astype(jnp.float32)
                rbase = sbase + r * (2 * WP)
                for j in range(24):
                    iy = lane2 + (rbase + 32 * j + (8 if j >= 12 else 0))
                    y = plsc.load_gather(span_buf, [iy])
                    x = plsc.load_gather(span_buf, [iy + 1])
                    cy = jnp.minimum(jnp.maximum(y + h_f, 0.0),
                                     jnp.float32(H - 1))
                    cx = jnp.minimum(jnp.maximum(x + (lane_f + (16.0 * j)),
                                                 0.0), jnp.float32(W - 1))
                    y0 = cy.astype(jnp.int32)
                    x0 = cx.astype(jnp.int32)
                    dy = cy - y0.astype(jnp.float32)
                    dx = cx - x0.astype(jnp.float32)
                    base = y0 * W + x0
                    lim = jnp.int32(HW - 1)

                    def corner(ci):
                        wd = plsc.load_gather(
                            map_buf, [lax.shift_right_logical(ci, 1)])
                        odd = (ci & 1) == 1
                        bits = jnp.where(odd, wd & jnp.int32(-65536),
                                         lax.shift_left(wd, 16))
                        return plsc.bitcast(bits, jnp.float32)

                    v00 = corner(base)
                    v01 = corner(jnp.minimum(base + 1, lim))
                    v10 = corner(jnp.minimum(base + W, lim))
                    v11 = corner(jnp.minimum(base + (W + 1), lim))
                    vt = dy * (v10 - v00) + v00
                    vb = dy * (v11 - v01) + v01
                    mp = dx * (vb - vt) + vt
                    sidx = lane + (ob + r * WP + (1 + 16 * j))
                    plsc.store_scatter(out_buf, [sidx], mp)
                return 0

            lax.fori_loop(0, _RB, row_body, 0)
            _out_copy(k).start()
            return 0

        lax.fori_loop(0, _KB, blk_body, 0)
        _out_copy(_KB - 2).wait()
        _out_copy(_KB - 1).wait()
        return 0

    lax.fori_loop(0, _MAPS_PER_TILE, map_body, 0)


def _sc_bilinear(offs_flat, xmap):
    mesh = plsc.VectorSubcoreMesh(core_axis_name="c", subcore_axis_name="s")
    f = pl.kernel(
        _sc_body,
        out_type=jax.ShapeDtypeStruct((C * MP,), jnp.float32),
        mesh=mesh,
        compiler_params=pltpu.CompilerParams(needs_layout_passes=False),
        scratch_types=[
            pltpu.VMEM((HW // 2,), jnp.int32),
            pltpu.VMEM((2 * _SPAN,), jnp.float32),
            pltpu.VMEM((2 * _OUTB,), jnp.float32),
            pltpu.VMEM((_OUTB,), jnp.float32),
            pltpu.SemaphoreType.DMA((2,)),
            pltpu.SemaphoreType.DMA((2,)),
        ],
    )
    return f(offs_flat, xmap)


def kernel(x, W_off, W_conv, b_conv):
    xp = jnp.pad(x, ((0, 0), (0, 0), (1, 15), (1, 7)))
    xp = xp.astype(jnp.bfloat16).reshape(B, C, MP)
    w1 = W_off.transpose(2, 3, 0, 1).reshape(9, 2 * C, C).astype(jnp.bfloat16)
    w2 = W_conv.transpose(2, 3, 0, 1).reshape(9, C, C).astype(jnp.bfloat16)
    b2 = b_conv.reshape(C, 1)
    xmap = lax.bitcast_convert_type(
        x.astype(jnp.bfloat16).reshape(NMAPS, HW // 2, 2),
        jnp.int32).reshape(NMAPS * (HW // 2))

    offs = [_conv1(xp, w1, b).reshape(2 * C * MOUT) for b in range(B)]
    xmap_b = [lax.slice(xmap, (b * C * (HW // 2),),
                        ((b + 1) * C * (HW // 2),)) for b in range(B)]
    mapped = [_sc_bilinear(offs[b], xmap_b[b]) for b in range(B)]
    outs = [_conv2(mapped[b].reshape(1, C, MP), w2, b2) for b in range(B)]
    return jnp.concatenate(outs, axis=0)
